# Initial kernel scaffold; baseline (speedup 1.0000x reference)
import jax
import jax.numpy as jnp
from jax.experimental import pallas as pl


def kernel(node_s, node_v, edge_s, edge_v, msg_Wh, msg_WV, msg_Ws_w, msg_Ws_b, msg_Wg_w, msg_Wg_b, ff0_Wh, ff0_WV, ff0_Ws_w, ff0_Ws_b, ff0_Wg_w, ff0_Wg_b, ff1_Wh, ff1_WV, ff1_Ws_w, ff1_Ws_b, ff1_Wg_w, ff1_Wg_b, ln1_w, ln1_b, ln2_w, ln2_b, edge_index):
    raise NotImplementedError("write your pallas kernel here")


# speedup vs baseline: 18.1039x; 18.1039x over previous
import functools

import jax
import jax.numpy as jnp
import numpy as np
from jax import lax
from jax.experimental import pallas as pl
from jax.experimental.pallas import tpu as pltpu
from jax.experimental.pallas import tpu_sc as plsc

F32 = jnp.float32

_N = 10000
_E = 320000
_NS = 128
_NP = 10240
_TW = 144
_SUB = 80
_NSUB = 5
_GROUP = _SUB * _NSUB
_NCORES = 2
_NTILES = 16
_NWORK = _NCORES * _NTILES
_EPW = _E // _NWORK
_NGRP = _EPW // _GROUP
_ROWS_PT = _NP // _NTILES
_SSUB = 40
_SNSUB = 5
_SGROUP = _SSUB * _SNSUB
_SNGRP = _EPW // _SGROUP
_BE = 2560
_BN = 1280


def _prep_body(ns_ref, nv_ref, a_ref, b_ref, dm_ref, wht_ref, wvt_ref,
               ts_ref, td_ref):
    ns = ns_ref[...]
    nv = nv_ref[...]
    nh = jnp.dot(nv, wht_ref[...], preferred_element_type=F32)
    anorm = jnp.sqrt(jnp.sum(nh * nh, axis=-1, keepdims=True))
    u = jnp.dot(nh, wvt_ref[...], preferred_element_type=F32)
    pa = jnp.dot(ns, a_ref[...], preferred_element_type=F32) + anorm * dm_ref[0:1, :]
    pb = jnp.dot(ns, b_ref[...], preferred_element_type=F32) + anorm * dm_ref[1:2, :]
    zpad = jnp.zeros((ns.shape[0], _TW - 131), F32)
    ts_ref[:, 0:128] = pa
    ts_ref[:, 128:131] = u
    ts_ref[:, 131:_TW] = zpad
    td_ref[:, 0:128] = pb
    td_ref[:, 128:131] = u
    td_ref[:, 131:_TW] = zpad


_prep_call = pl.pallas_call(
    _prep_body,
    grid=(_NP // _BN,),
    in_specs=[
        pl.BlockSpec((_BN, _NS), lambda i: (i, 0)),
        pl.BlockSpec((_BN, 3), lambda i: (i, 0)),
        pl.BlockSpec((_NS, _NS), lambda i: (0, 0)),
        pl.BlockSpec((_NS, _NS), lambda i: (0, 0)),
        pl.BlockSpec((3, _NS), lambda i: (0, 0)),
        pl.BlockSpec((3, 3), lambda i: (0, 0)),
        pl.BlockSpec((3, 3), lambda i: (0, 0)),
    ],
    out_specs=[pl.BlockSpec((_BN, _TW), lambda i: (i, 0))] * 2,
    out_shape=[jax.ShapeDtypeStruct((_NP, _TW), F32)] * 2,
)


def _gather_body(ts_hbm, td_hbm, si2_hbm, di2_hbm, gs_hbm, gd_hbm,
                 sidx, didx, bs, bd, sem):
    c = lax.axis_index("c")
    s = lax.axis_index("s")
    wid = s * _NCORES + c

    def body(g, carry):
        base = wid * _EPW + g * _GROUP
        gid = wid * _NGRP + g
        pltpu.sync_copy(si2_hbm.at[gid], sidx)
        pltpu.sync_copy(di2_hbm.at[gid], didx)
        cps = []
        for j in range(_NSUB):
            cps.append(pltpu.async_copy(
                ts_hbm.at[sidx.at[j]], bs.at[pl.ds(j * _SUB, _SUB)], sem))
            cps.append(pltpu.async_copy(
                td_hbm.at[didx.at[j]], bd.at[pl.ds(j * _SUB, _SUB)], sem))
        for cp in cps:
            cp.wait()
        pltpu.sync_copy(bs, gs_hbm.at[pl.ds(base, _GROUP)])
        pltpu.sync_copy(bd, gd_hbm.at[pl.ds(base, _GROUP)])
        return carry

    lax.fori_loop(0, _NGRP, body, 0)


@functools.cache
def _gather_call():
    return pl.kernel(
        _gather_body,
        out_type=(jax.ShapeDtypeStruct((_E, _TW), F32),
                  jax.ShapeDtypeStruct((_E, _TW), F32)),
        mesh=plsc.VectorSubcoreMesh(core_axis_name="c", subcore_axis_name="s",
                                    num_cores=_NCORES, num_subcores=_NTILES),
        scratch_types=[
            pltpu.VMEM((_NSUB, _SUB), jnp.int32),
            pltpu.VMEM((_NSUB, _SUB), jnp.int32),
            pltpu.VMEM((_GROUP, _TW), F32),
            pltpu.VMEM((_GROUP, _TW), F32),
            pltpu.SemaphoreType.DMA,
        ],
        compiler_params=pltpu.CompilerParams(use_tc_tiling_on_sc=False),
    )


def _edge_body(gs_ref, gd_ref, es_ref, ev_ref, c16_ref, dm2_ref, bias_ref,
               wg_ref, wgb_ref, wht_ref, wc_ref, m_ref):
    ev = ev_ref[...]
    vh = jnp.dot(ev, wht_ref[...], preferred_element_type=F32)
    cnorm = jnp.sqrt(jnp.sum(vh * vh, axis=-1, keepdims=True))
    evp = jnp.dot(ev, wc_ref[...], preferred_element_type=F32)
    q = jnp.dot(es_ref[...], c16_ref[...], preferred_element_type=F32)
    slin = (gs_ref[:, 0:128] + gd_ref[:, 0:128] + q
            + cnorm * dm2_ref[...] + bias_ref[...])
    so = jnp.maximum(slin, 0.0)
    d0 = jnp.sum(so * wg_ref[0:1, :], axis=-1, keepdims=True)
    d1 = jnp.sum(so * wg_ref[1:2, :], axis=-1, keepdims=True)
    d2 = jnp.sum(so * wg_ref[2:3, :], axis=-1, keepdims=True)
    gate = jax.nn.sigmoid(jnp.concatenate([d0, d1, d2], axis=1) + wgb_ref[...])
    r0 = gate[:, 0:1] * gs_ref[:, 128:131]
    r1 = gate[:, 1:2] * gd_ref[:, 128:131]
    r2 = gate[:, 2:3] * evp
    m_ref[:, 0:128] = so
    m_ref[:, 128:137] = jnp.concatenate([r0, r1, r2], axis=1)
    m_ref[:, 137:_TW] = jnp.zeros((so.shape[0], _TW - 137), F32)


_edge_call = pl.pallas_call(
    _edge_body,
    grid=(_E // _BE,),
    in_specs=[
        pl.BlockSpec((_BE, _TW), lambda i: (i, 0)),
        pl.BlockSpec((_BE, _TW), lambda i: (i, 0)),
        pl.BlockSpec((_BE, 16), lambda i: (i, 0)),
        pl.BlockSpec((_BE, 3), lambda i: (i, 0)),
        pl.BlockSpec((16, _NS), lambda i: (0, 0)),
        pl.BlockSpec((1, _NS), lambda i: (0, 0)),
        pl.BlockSpec((1, _NS), lambda i: (0, 0)),
        pl.BlockSpec((3, _NS), lambda i: (0, 0)),
        pl.BlockSpec((1, 3), lambda i: (0, 0)),
        pl.BlockSpec((3, 3), lambda i: (0, 0)),
        pl.BlockSpec((3, 3), lambda i: (0, 0)),
    ],
    out_specs=pl.BlockSpec((_BE, _TW), lambda i: (i, 0)),
    out_shape=jax.ShapeDtypeStruct((_E, _TW), F32),
)


def _scatter_body(m_hbm, di2_hbm, z_hbm, out_hbm, didx, buf, acc, sem):
    c = lax.axis_index("c")
    s = lax.axis_index("s")
    pltpu.sync_copy(z_hbm.at[pl.ds(s * _ROWS_PT, _ROWS_PT)],
                    acc.at[pl.ds(s * _ROWS_PT, _ROWS_PT)])
    plsc.subcore_barrier()
    base0 = c * (_E // _NCORES) + s * _EPW

    def body(g, carry):
        base = base0 + g * _SGROUP
        gid = base0 // _SGROUP + g
        pltpu.sync_copy(di2_hbm.at[gid], didx)
        pltpu.sync_copy(m_hbm.at[pl.ds(base, _SGROUP)], buf)
        cps = []
        for j in range(_SNSUB):
            cps.append(pltpu.async_copy(
                buf.at[pl.ds(j * _SSUB, _SSUB)], acc.at[didx.at[j]], sem,
                add=True))
        for cp in cps:
            cp.wait()
        return carry

    lax.fori_loop(0, _SNGRP, body, 0)
    plsc.subcore_barrier()
    pltpu.sync_copy(acc.at[pl.ds(s * _ROWS_PT, _ROWS_PT)],
                    out_hbm.at[c, pl.ds(s * _ROWS_PT, _ROWS_PT)])


@functools.cache
def _scatter_call():
    return pl.kernel(
        _scatter_body,
        out_type=jax.ShapeDtypeStruct((_NCORES, _NP, _TW), F32),
        mesh=plsc.VectorSubcoreMesh(core_axis_name="c", subcore_axis_name="s",
                                    num_cores=_NCORES, num_subcores=_NTILES),
        scratch_types=[
            pltpu.VMEM((_SNSUB, _SSUB), jnp.int32),
            pltpu.VMEM((_SGROUP, _TW), F32),
            pltpu.VMEM_SHARED((_NP, _TW), F32),
            pltpu.SemaphoreType.DMA,
        ],
        compiler_params=pltpu.CompilerParams(use_tc_tiling_on_sc=False),
    )


def _node_gvp(s, v9, a_ref, d_ref, b_ref, wg_ref, wgb_ref, bdh_ref, bdc_ref,
              sel_ref, selt_ref):
    vh9 = jnp.dot(v9, bdh_ref[...], preferred_element_type=F32)
    vn = jnp.sqrt(jnp.dot(vh9 * vh9, sel_ref[...],
                          preferred_element_type=F32))
    slin = (jnp.dot(s, a_ref[...], preferred_element_type=F32)
            + jnp.dot(vn, d_ref[...], preferred_element_type=F32)
            + b_ref[...])
    so = jnp.maximum(slin, 0.0)
    d0 = jnp.sum(so * wg_ref[0:1, :], axis=-1, keepdims=True)
    d1 = jnp.sum(so * wg_ref[1:2, :], axis=-1, keepdims=True)
    d2 = jnp.sum(so * wg_ref[2:3, :], axis=-1, keepdims=True)
    gate = jax.nn.sigmoid(jnp.concatenate([d0, d1, d2], axis=1) + wgb_ref[...])
    gate9 = jnp.dot(gate, selt_ref[...], preferred_element_type=F32)
    vout = jnp.dot(v9, bdc_ref[...], preferred_element_type=F32) * gate9
    return so, vout


def _layernorm(x, w, b):
    mu = jnp.mean(x, axis=-1, keepdims=True)
    var = jnp.mean((x - mu) ** 2, axis=-1, keepdims=True)
    return (x - mu) / jnp.sqrt(var + 1e-5) * w + b


def _node_body(p0_ref, p1_ref, ns_ref,
               ln1w_ref, ln1b_ref, ln2w_ref, ln2b_ref,
               a0_ref, d0_ref, b0_ref, wg0_ref, wgb0_ref, bdh0_ref, bdc0_ref,
               a1_ref, d1_ref, b1_ref, wg1_ref, wgb1_ref, bdh1_ref, bdc1_ref,
               sel_ref, selt_ref, os_ref, ov_ref):
    agg_s = p0_ref[:, 0:128] + p1_ref[:, 0:128] + ns_ref[...]
    v9 = p0_ref[:, 128:137] + p1_ref[:, 128:137]
    s1 = _layernorm(agg_s, ln1w_ref[...], ln1b_ref[...])
    rms = jnp.sqrt(jnp.mean(v9 * v9, axis=-1, keepdims=True) + 1e-8)
    v1 = v9 / rms
    s2, v2 = _node_gvp(s1, v1, a0_ref, d0_ref, b0_ref, wg0_ref, wgb0_ref,
                       bdh0_ref, bdc0_ref, sel_ref, selt_ref)
    s3, v3 = _node_gvp(s2, v2, a1_ref, d1_ref, b1_ref, wg1_ref, wgb1_ref,
                       bdh1_ref, bdc1_ref, sel_ref, selt_ref)
    o_s = s1 + s3
    o_v = v1 + v3
    os_ref[...] = _layernorm(o_s, ln2w_ref[...], ln2b_ref[...])
    rms2 = jnp.sqrt(jnp.mean(o_v * o_v, axis=-1, keepdims=True) + 1e-8)
    ov_ref[...] = o_v / rms2


def _full(shape):
    return pl.BlockSpec(shape, lambda i: tuple(0 for _ in shape))


_node_call = pl.pallas_call(
    _node_body,
    grid=(_NP // _BN,),
    in_specs=[
        pl.BlockSpec((_BN, _TW), lambda i: (i, 0)),
        pl.BlockSpec((_BN, _TW), lambda i: (i, 0)),
        pl.BlockSpec((_BN, _NS), lambda i: (i, 0)),
        _full((1, _NS)), _full((1, _NS)), _full((1, _NS)), _full((1, _NS)),
        _full((_NS, _NS)), _full((3, _NS)), _full((1, _NS)),
        _full((3, _NS)), _full((1, 3)), _full((9, 9)), _full((9, 9)),
        _full((_NS, _NS)), _full((3, _NS)), _full((1, _NS)),
        _full((3, _NS)), _full((1, 3)), _full((9, 9)), _full((9, 9)),
        _full((9, 3)), _full((3, 9)),
    ],
    out_specs=[pl.BlockSpec((_BN, _NS), lambda i: (i, 0)),
               pl.BlockSpec((_BN, 9), lambda i: (i, 0))],
    out_shape=[jax.ShapeDtypeStruct((_NP, _NS), F32),
               jax.ShapeDtypeStruct((_NP, 9), F32)],
)

_SEL = np.zeros((9, 3), np.float32)
for _i in range(3):
    for _k in range(3):
        _SEL[3 * _i + _k, _i] = 1.0


def _blockdiag3(w):
    z = jnp.zeros((9, 9), F32)
    for i in range(3):
        z = z.at[3 * i:3 * i + 3, 3 * i:3 * i + 3].set(w)
    return z


def kernel(node_s, node_v, edge_s, edge_v, msg_Wh, msg_WV, msg_Ws_w,
           msg_Ws_b, msg_Wg_w, msg_Wg_b, ff0_Wh, ff0_WV, ff0_Ws_w, ff0_Ws_b,
           ff0_Wg_w, ff0_Wg_b, ff1_Wh, ff1_WV, ff1_Ws_w, ff1_Ws_b, ff1_Wg_w,
           ff1_Wg_b, ln1_w, ln1_b, ln2_w, ln2_b, edge_index):
    ns_p = jnp.zeros((_NP, _NS), F32).at[:_N].set(node_s)
    nv_p = jnp.zeros((_NP, 3), F32).at[:_N].set(node_v.reshape(_N, 3))
    wst = msg_Ws_w.T
    a_w, b_w, c16, dm = wst[0:128], wst[128:256], wst[256:272], wst[272:275]
    wht = msg_Wh.T
    wc = msg_Wh.T @ msg_WV.T
    ts, td = _prep_call(ns_p, nv_p, a_w, b_w, dm, wht, msg_WV.T)
    src2 = edge_index[0].reshape(_E // _GROUP, _NSUB, _SUB)
    dst2 = edge_index[1].reshape(_E // _GROUP, _NSUB, _SUB)
    gs, gd = _gather_call()(ts, td, src2, dst2)
    ev3 = edge_v.reshape(_E, 3)
    m = _edge_call(gs, gd, edge_s, ev3, c16, dm[2:3], msg_Ws_b[None],
                   msg_Wg_w, msg_Wg_b[None], wht, wc)
    zeros = jnp.zeros((_NP, _TW), F32)
    dst2s = edge_index[1].reshape(_E // _SGROUP, _SNSUB, _SSUB)
    parts = _scatter_call()(m, dst2s, zeros)

    def ffw(ws_w, ws_b, wg_w, wg_b, wh, wv):
        t = ws_w.T
        return (t[0:128], t[128:131], ws_b[None], wg_w, wg_b[None],
                _blockdiag3(wh.T), _blockdiag3(wh.T @ wv.T))

    sel = jnp.asarray(_SEL)
    out_s, out_v9 = _node_call(
        parts[0], parts[1], ns_p,
        ln1_w[None], ln1_b[None], ln2_w[None], ln2_b[None],
        *ffw(ff0_Ws_w, ff0_Ws_b, ff0_Wg_w, ff0_Wg_b, ff0_Wh, ff0_WV),
        *ffw(ff1_Ws_w, ff1_Ws_b, ff1_Wg_w, ff1_Wg_b, ff1_Wh, ff1_WV),
        sel, sel.T)
    return out_s[:_N], out_v9[:_N].reshape(_N, 3, 3)


# speedup vs baseline: 30.1761x; 1.6668x over previous
import functools

import jax
import jax.numpy as jnp
import numpy as np
from jax import lax
from jax.experimental import pallas as pl
from jax.experimental.pallas import tpu as pltpu
from jax.experimental.pallas import tpu_sc as plsc

F32 = jnp.float32

_N = 10000
_E = 320000
_NS = 128
_NP = 10240
_VW = 16
_SUB = 80
_NSUB = 5
_GROUP = _SUB * _NSUB
_NCORES = 2
_NTILES = 16
_NWORK = _NCORES * _NTILES
_EPW = _E // _NWORK
_NGRP = _EPW // _GROUP
_ROWS_PT = _NP // _NTILES
_SSUB = 40
_SNSUB = 5
_SGROUP = _SSUB * _SNSUB
_SNGRP = _EPW // _SGROUP
_BE = 2560
_BN = 1280


def _prep_body(ns_ref, nv_ref, a_ref, b_ref, dm_ref, wht_ref, wvt_ref,
               ts_ref, td_ref, tu_ref):
    ns = ns_ref[...]
    nv = nv_ref[...]
    nh = jnp.dot(nv, wht_ref[...], preferred_element_type=F32)
    anorm = jnp.sqrt(jnp.sum(nh * nh, axis=-1, keepdims=True))
    u = jnp.dot(nh, wvt_ref[...], preferred_element_type=F32)
    ts_ref[...] = jnp.dot(ns, a_ref[...], preferred_element_type=F32) \
        + anorm * dm_ref[0:1, :]
    td_ref[...] = jnp.dot(ns, b_ref[...], preferred_element_type=F32) \
        + anorm * dm_ref[1:2, :]
    tu_ref[:, 0:3] = u
    tu_ref[:, 3:_VW] = jnp.zeros((ns.shape[0], _VW - 3), F32)


_prep_call = pl.pallas_call(
    _prep_body,
    grid=(_NP // _BN,),
    in_specs=[
        pl.BlockSpec((_BN, _NS), lambda i: (i, 0)),
        pl.BlockSpec((_BN, 3), lambda i: (i, 0)),
        pl.BlockSpec((_NS, _NS), lambda i: (0, 0)),
        pl.BlockSpec((_NS, _NS), lambda i: (0, 0)),
        pl.BlockSpec((3, _NS), lambda i: (0, 0)),
        pl.BlockSpec((3, 3), lambda i: (0, 0)),
        pl.BlockSpec((3, 3), lambda i: (0, 0)),
    ],
    out_specs=[pl.BlockSpec((_BN, _NS), lambda i: (i, 0)),
               pl.BlockSpec((_BN, _NS), lambda i: (i, 0)),
               pl.BlockSpec((_BN, _VW), lambda i: (i, 0))],
    out_shape=[jax.ShapeDtypeStruct((_NP, _NS), F32),
               jax.ShapeDtypeStruct((_NP, _NS), F32),
               jax.ShapeDtypeStruct((_NP, _VW), F32)],
)


def _gather1_body(ts_hbm, td_hbm, si3_hbm, di3_hbm, gs_hbm, gd_hbm,
                  sidx, didx, bs, bd, sem):
    c = lax.axis_index("c")
    s = lax.axis_index("s")
    wid = s * _NCORES + c

    def body(g, carry):
        base = wid * _EPW + g * _GROUP
        gid = wid * _NGRP + g
        pltpu.sync_copy(si3_hbm.at[gid], sidx)
        pltpu.sync_copy(di3_hbm.at[gid], didx)
        cps = []
        for j in range(_NSUB):
            cps.append(pltpu.async_copy(
                ts_hbm.at[sidx.at[j]], bs.at[pl.ds(j * _SUB, _SUB)], sem))
            cps.append(pltpu.async_copy(
                td_hbm.at[didx.at[j]], bd.at[pl.ds(j * _SUB, _SUB)], sem))
        for cp in cps:
            cp.wait()
        pltpu.sync_copy(bs, gs_hbm.at[pl.ds(base, _GROUP)])
        pltpu.sync_copy(bd, gd_hbm.at[pl.ds(base, _GROUP)])
        return carry

    lax.fori_loop(0, _NGRP, body, 0)


@functools.cache
def _gather1_call():
    return pl.kernel(
        _gather1_body,
        out_type=(jax.ShapeDtypeStruct((_E, _NS), F32),
                  jax.ShapeDtypeStruct((_E, _NS), F32)),
        mesh=plsc.VectorSubcoreMesh(core_axis_name="c", subcore_axis_name="s",
                                    num_cores=_NCORES, num_subcores=_NTILES),
        scratch_types=[
            pltpu.VMEM((_NSUB, _SUB), jnp.int32),
            pltpu.VMEM((_NSUB, _SUB), jnp.int32),
            pltpu.VMEM((_GROUP, _NS), F32),
            pltpu.VMEM((_GROUP, _NS), F32),
            pltpu.SemaphoreType.DMA,
        ],
    )


def _gather2_body(tu_hbm, si3_hbm, us_hbm, sidx, bu, sem):
    c = lax.axis_index("c")
    s = lax.axis_index("s")
    wid = s * _NCORES + c

    def body(g, carry):
        base = wid * _EPW + g * _GROUP
        gid = wid * _NGRP + g
        pltpu.sync_copy(si3_hbm.at[gid], sidx)
        cps = []
        for j in range(_NSUB):
            cps.append(pltpu.async_copy(
                tu_hbm.at[sidx.at[j]], bu.at[pl.ds(j * _SUB, _SUB)], sem))
        for cp in cps:
            cp.wait()
        pltpu.sync_copy(bu, us_hbm.at[pl.ds(base, _GROUP)])
        return carry

    lax.fori_loop(0, _NGRP, body, 0)


@functools.cache
def _gather2_call():
    return pl.kernel(
        _gather2_body,
        out_type=jax.ShapeDtypeStruct((_E, _VW), F32),
        mesh=plsc.VectorSubcoreMesh(core_axis_name="c", subcore_axis_name="s",
                                    num_cores=_NCORES, num_subcores=_NTILES),
        scratch_types=[
            pltpu.VMEM((_NSUB, _SUB), jnp.int32),
            pltpu.VMEM((_GROUP, _VW), F32),
            pltpu.SemaphoreType.DMA,
        ],
        compiler_params=pltpu.CompilerParams(use_tc_tiling_on_sc=False),
    )


def _edge_body(gs_ref, gd_ref, us_ref, es_ref, ev_ref, c16_ref, dm2_ref,
               bias_ref, wg8_ref, wgb8_ref, wht_ref, wc_ref,
               ms_ref, mv_ref):
    ev = ev_ref[...]
    vh = jnp.dot(ev, wht_ref[...], preferred_element_type=F32)
    cnorm = jnp.sqrt(jnp.sum(vh * vh, axis=-1, keepdims=True))
    evp = jnp.dot(ev, wc_ref[...], preferred_element_type=F32)
    q = jnp.dot(es_ref[...], c16_ref[...], preferred_element_type=F32)
    slin = (gs_ref[...] + gd_ref[...] + q
            + cnorm * dm2_ref[...] + bias_ref[...])
    so = jnp.maximum(slin, 0.0)
    gate = jax.nn.sigmoid(
        jnp.dot(so, wg8_ref[...], preferred_element_type=F32) + wgb8_ref[...])
    r0 = gate[:, 0:1] * us_ref[:, 0:3]
    r2 = gate[:, 2:3] * evp
    ms_ref[...] = so
    mv_ref[:, 0:3] = r0
    mv_ref[:, 3:4] = gate[:, 1:2]
    mv_ref[:, 4:7] = r2
    mv_ref[:, 7:_VW] = jnp.zeros((so.shape[0], _VW - 7), F32)


_edge_call = pl.pallas_call(
    _edge_body,
    grid=(_E // _BE,),
    in_specs=[
        pl.BlockSpec((_BE, _NS), lambda i: (i, 0)),
        pl.BlockSpec((_BE, _NS), lambda i: (i, 0)),
        pl.BlockSpec((_BE, _VW), lambda i: (i, 0)),
        pl.BlockSpec((_BE, 16), lambda i: (i, 0)),
        pl.BlockSpec((_BE, 3), lambda i: (i, 0)),
        pl.BlockSpec((16, _NS), lambda i: (0, 0)),
        pl.BlockSpec((1, _NS), lambda i: (0, 0)),
        pl.BlockSpec((1, _NS), lambda i: (0, 0)),
        pl.BlockSpec((_NS, 8), lambda i: (0, 0)),
        pl.BlockSpec((1, 8), lambda i: (0, 0)),
        pl.BlockSpec((3, 3), lambda i: (0, 0)),
        pl.BlockSpec((3, 3), lambda i: (0, 0)),
    ],
    out_specs=[pl.BlockSpec((_BE, _NS), lambda i: (i, 0)),
               pl.BlockSpec((_BE, _VW), lambda i: (i, 0))],
    out_shape=[jax.ShapeDtypeStruct((_E, _NS), F32),
               jax.ShapeDtypeStruct((_E, _VW), F32)],
)


def _scatter1_body(m_hbm, di3_hbm, z_hbm, out_hbm, didx, buf, acc, sem):
    c = lax.axis_index("c")
    s = lax.axis_index("s")
    pltpu.sync_copy(z_hbm.at[pl.ds(s * _ROWS_PT, _ROWS_PT)],
                    acc.at[pl.ds(s * _ROWS_PT, _ROWS_PT)])
    plsc.subcore_barrier()
    base0 = c * (_E // _NCORES) + s * _EPW

    def body(g, carry):
        base = base0 + g * _SGROUP
        gid = base0 // _SGROUP + g
        pltpu.sync_copy(di3_hbm.at[gid], didx)
        pltpu.sync_copy(m_hbm.at[pl.ds(base, _SGROUP)], buf)
        cps = []
        for j in range(_SNSUB):
            cps.append(pltpu.async_copy(
                buf.at[pl.ds(j * _SSUB, _SSUB)], acc.at[didx.at[j]], sem,
                add=True))
        for cp in cps:
            cp.wait()
        return carry

    lax.fori_loop(0, _SNGRP, body, 0)
    plsc.subcore_barrier()
    pltpu.sync_copy(acc.at[pl.ds(s * _ROWS_PT, _ROWS_PT)],
                    out_hbm.at[c, pl.ds(s * _ROWS_PT, _ROWS_PT)])


@functools.cache
def _scatter1_call():
    return pl.kernel(
        _scatter1_body,
        out_type=jax.ShapeDtypeStruct((_NCORES, _NP, _NS), F32),
        mesh=plsc.VectorSubcoreMesh(core_axis_name="c", subcore_axis_name="s",
                                    num_cores=_NCORES, num_subcores=_NTILES),
        scratch_types=[
            pltpu.VMEM((_SNSUB, _SSUB), jnp.int32),
            pltpu.VMEM((_SGROUP, _NS), F32),
            pltpu.VMEM_SHARED((_NP, _NS), F32),
            pltpu.SemaphoreType.DMA,
        ],
    )


def _scatter2_body(m_hbm, di3_hbm, z_hbm, out_hbm, didx, buf, acc, sem):
    c = lax.axis_index("c")
    s = lax.axis_index("s")
    pltpu.sync_copy(z_hbm.at[pl.ds(s * _ROWS_PT, _ROWS_PT)],
                    acc.at[pl.ds(s * _ROWS_PT, _ROWS_PT)])
    plsc.subcore_barrier()
    base0 = c * (_E // _NCORES) + s * _EPW

    def body(g, carry):
        base = base0 + g * _GROUP
        gid = base0 // _GROUP + g
        pltpu.sync_copy(di3_hbm.at[gid], didx)
        pltpu.sync_copy(m_hbm.at[pl.ds(base, _GROUP)], buf)
        cps = []
        for j in range(_NSUB):
            cps.append(pltpu.async_copy(
                buf.at[pl.ds(j * _SUB, _SUB)], acc.at[didx.at[j]], sem,
                add=True))
        for cp in cps:
            cp.wait()
        return carry

    lax.fori_loop(0, _NGRP, body, 0)
    plsc.subcore_barrier()
    pltpu.sync_copy(acc.at[pl.ds(s * _ROWS_PT, _ROWS_PT)],
                    out_hbm.at[c, pl.ds(s * _ROWS_PT, _ROWS_PT)])


@functools.cache
def _scatter2_call():
    return pl.kernel(
        _scatter2_body,
        out_type=jax.ShapeDtypeStruct((_NCORES, _NP, _VW), F32),
        mesh=plsc.VectorSubcoreMesh(core_axis_name="c", subcore_axis_name="s",
                                    num_cores=_NCORES, num_subcores=_NTILES),
        scratch_types=[
            pltpu.VMEM((_NSUB, _SUB), jnp.int32),
            pltpu.VMEM((_GROUP, _VW), F32),
            pltpu.VMEM_SHARED((_NP, _VW), F32),
            pltpu.SemaphoreType.DMA,
        ],
        compiler_params=pltpu.CompilerParams(use_tc_tiling_on_sc=False),
    )


def _node_gvp(s, v9, a_ref, d_ref, b_ref, wg8_ref, wgb8_ref, bdh_ref,
              bdc_ref, sel_ref, selt_ref):
    vh9 = jnp.dot(v9, bdh_ref[...], preferred_element_type=F32)
    vn = jnp.sqrt(jnp.dot(vh9 * vh9, sel_ref[...],
                          preferred_element_type=F32))
    slin = (jnp.dot(s, a_ref[...], preferred_element_type=F32)
            + jnp.dot(vn, d_ref[...], preferred_element_type=F32)
            + b_ref[...])
    so = jnp.maximum(slin, 0.0)
    gate = jax.nn.sigmoid(
        jnp.dot(so, wg8_ref[...], preferred_element_type=F32)
        + wgb8_ref[...])[:, 0:3]
    gate9 = jnp.dot(gate, selt_ref[...], preferred_element_type=F32)
    vout = jnp.dot(v9, bdc_ref[...], preferred_element_type=F32) * gate9
    return so, vout


def _layernorm(x, w, b):
    mu = jnp.mean(x, axis=-1, keepdims=True)
    var = jnp.mean((x - mu) ** 2, axis=-1, keepdims=True)
    return (x - mu) / jnp.sqrt(var + 1e-5) * w + b


def _node_body(p0_ref, p1_ref, pv0_ref, pv1_ref, ns_ref, nv_ref,
               wht_ref, wvt_ref,
               ln1w_ref, ln1b_ref, ln2w_ref, ln2b_ref,
               a0_ref, d0_ref, b0_ref, wg0_ref, wgb0_ref, bdh0_ref, bdc0_ref,
               a1_ref, d1_ref, b1_ref, wg1_ref, wgb1_ref, bdh1_ref, bdc1_ref,
               sel_ref, selt_ref, os_ref, ov_ref):
    agg_s = p0_ref[...] + p1_ref[...] + ns_ref[...]
    pv = pv0_ref[...] + pv1_ref[...]
    nh = jnp.dot(nv_ref[...], wht_ref[...], preferred_element_type=F32)
    u = jnp.dot(nh, wvt_ref[...], preferred_element_type=F32)
    v9 = jnp.concatenate(
        [pv[:, 0:3], u * pv[:, 3:4], pv[:, 4:7]], axis=1)
    s1 = _layernorm(agg_s, ln1w_ref[...], ln1b_ref[...])
    rms = jnp.sqrt(jnp.mean(v9 * v9, axis=-1, keepdims=True) + 1e-8)
    v1 = v9 / rms
    s2, v2 = _node_gvp(s1, v1, a0_ref, d0_ref, b0_ref, wg0_ref, wgb0_ref,
                       bdh0_ref, bdc0_ref, sel_ref, selt_ref)
    s3, v3 = _node_gvp(s2, v2, a1_ref, d1_ref, b1_ref, wg1_ref, wgb1_ref,
                       bdh1_ref, bdc1_ref, sel_ref, selt_ref)
    o_s = s1 + s3
    o_v = v1 + v3
    os_ref[...] = _layernorm(o_s, ln2w_ref[...], ln2b_ref[...])
    rms2 = jnp.sqrt(jnp.mean(o_v * o_v, axis=-1, keepdims=True) + 1e-8)
    ov_ref[...] = o_v / rms2


def _full(shape):
    return pl.BlockSpec(shape, lambda i: tuple(0 for _ in shape))


_node_call = pl.pallas_call(
    _node_body,
    grid=(_NP // _BN,),
    in_specs=[
        pl.BlockSpec((_BN, _NS), lambda i: (i, 0)),
        pl.BlockSpec((_BN, _NS), lambda i: (i, 0)),
        pl.BlockSpec((_BN, _VW), lambda i: (i, 0)),
        pl.BlockSpec((_BN, _VW), lambda i: (i, 0)),
        pl.BlockSpec((_BN, _NS), lambda i: (i, 0)),
        pl.BlockSpec((_BN, 3), lambda i: (i, 0)),
        _full((3, 3)), _full((3, 3)),
        _full((1, _NS)), _full((1, _NS)), _full((1, _NS)), _full((1, _NS)),
        _full((_NS, _NS)), _full((3, _NS)), _full((1, _NS)),
        _full((_NS, 8)), _full((1, 8)), _full((9, 9)), _full((9, 9)),
        _full((_NS, _NS)), _full((3, _NS)), _full((1, _NS)),
        _full((_NS, 8)), _full((1, 8)), _full((9, 9)), _full((9, 9)),
        _full((9, 3)), _full((3, 9)),
    ],
    out_specs=[pl.BlockSpec((_BN, _NS), lambda i: (i, 0)),
               pl.BlockSpec((_BN, 9), lambda i: (i, 0))],
    out_shape=[jax.ShapeDtypeStruct((_NP, _NS), F32),
               jax.ShapeDtypeStruct((_NP, 9), F32)],
)

_SEL = np.zeros((9, 3), np.float32)
for _i in range(3):
    for _k in range(3):
        _SEL[3 * _i + _k, _i] = 1.0


def _blockdiag3(w):
    z = jnp.zeros((9, 9), F32)
    for i in range(3):
        z = z.at[3 * i:3 * i + 3, 3 * i:3 * i + 3].set(w)
    return z


def _pad8(w3):
    return jnp.zeros((w3.shape[1], 8), F32).at[:, 0:3].set(w3.T)


def kernel(node_s, node_v, edge_s, edge_v, msg_Wh, msg_WV, msg_Ws_w,
           msg_Ws_b, msg_Wg_w, msg_Wg_b, ff0_Wh, ff0_WV, ff0_Ws_w, ff0_Ws_b,
           ff0_Wg_w, ff0_Wg_b, ff1_Wh, ff1_WV, ff1_Ws_w, ff1_Ws_b, ff1_Wg_w,
           ff1_Wg_b, ln1_w, ln1_b, ln2_w, ln2_b, edge_index):
    ns_p = jnp.zeros((_NP, _NS), F32).at[:_N].set(node_s)
    nv_p = jnp.zeros((_NP, 3), F32).at[:_N].set(node_v.reshape(_N, 3))
    wst = msg_Ws_w.T
    a_w, b_w, c16, dm = wst[0:128], wst[128:256], wst[256:272], wst[272:275]
    wht = msg_Wh.T
    wc = msg_Wh.T @ msg_WV.T
    ts, td, tu = _prep_call(ns_p, nv_p, a_w, b_w, dm, wht, msg_WV.T)
    src3 = edge_index[0].reshape(_E // _GROUP, _NSUB, _SUB)
    dst3 = edge_index[1].reshape(_E // _GROUP, _NSUB, _SUB)
    gs, gd = _gather1_call()(ts, td, src3, dst3)
    us = _gather2_call()(tu, src3)
    ev3 = edge_v.reshape(_E, 3)
    wgb8 = jnp.zeros((1, 8), F32).at[0, 0:3].set(msg_Wg_b)
    m_s, m_v = _edge_call(gs, gd, us, edge_s, ev3, c16, dm[2:3],
                          msg_Ws_b[None], _pad8(msg_Wg_w), wgb8, wht, wc)
    zeros_s = jnp.zeros((_NP, _NS), F32)
    zeros_v = jnp.zeros((_NP, _VW), F32)
    dst3s = edge_index[1].reshape(_E // _SGROUP, _SNSUB, _SSUB)
    parts_s = _scatter1_call()(m_s, dst3s, zeros_s)
    parts_v = _scatter2_call()(m_v, dst3, zeros_v)

    def ffw(ws_w, ws_b, wg_w, wg_b, wh, wv):
        t = ws_w.T
        wgb = jnp.zeros((1, 8), F32).at[0, 0:3].set(wg_b)
        return (t[0:128], t[128:131], ws_b[None], _pad8(wg_w), wgb,
                _blockdiag3(wh.T), _blockdiag3(wh.T @ wv.T))

    sel = jnp.asarray(_SEL)
    out_s, out_v9 = _node_call(
        parts_s[0], parts_s[1], parts_v[0], parts_v[1], ns_p, nv_p,
        wht, msg_WV.T,
        ln1_w[None], ln1_b[None], ln2_w[None], ln2_b[None],
        *ffw(ff0_Ws_w, ff0_Ws_b, ff0_Wg_w, ff0_Wg_b, ff0_Wh, ff0_WV),
        *ffw(ff1_Ws_w, ff1_Ws_b, ff1_Wg_w, ff1_Wg_b, ff1_Wh, ff1_WV),
        sel, sel.T)
    return out_s[:_N], out_v9[:_N].reshape(_N, 3, 3)
